# Initial kernel scaffold; baseline (speedup 1.0000x reference)
import functools

import jax
import jax.numpy as jnp
from jax.experimental import pallas as pl
from jax.experimental.pallas import tpu as pltpu

_K = 1024
_D = 64
_TB = 512


def _vq_block(x_ref, w_ref, loss_ref, q_ref, perp_ref, enc_ref,
              loss_acc, hist_acc, *, nblocks, n_tokens):
    i = pl.program_id(0)

    @pl.when(i == 0)
    def _init():
        loss_acc[0, 0] = 0.0
        hist_acc[...] = jnp.zeros_like(hist_acc)

    x = x_ref[...]
    w = w_ref[...]
    x_sq = jnp.sum(x * x, axis=1, keepdims=True)
    w_sq = jnp.sum(w * w, axis=1)
    xw = jax.lax.dot_general(x, w, (((1,), (1,)), ((), ())))
    dist = x_sq + w_sq[None, :] - 2.0 * xw
    idx = jnp.argmin(dist, axis=1)
    iota = jax.lax.broadcasted_iota(jnp.int32, (x.shape[0], _K), 1)
    one_hot = (iota == idx[:, None]).astype(jnp.float32)
    enc_ref[...] = one_hot
    q = jax.lax.dot_general(one_hot, w, (((1,), (0,)), ((), ())))
    q_ref[...] = q
    d = q - x
    loss_acc[0, 0] += jnp.sum(d * d)
    hist_acc[...] += jnp.sum(one_hot, axis=0, keepdims=True)

    @pl.when(i == nblocks - 1)
    def _fin():
        m = loss_acc[0, 0] / (n_tokens * _D)
        loss_ref[0, 0] = m + 0.25 * m
        p = hist_acc[...] / n_tokens
        perp_ref[0, 0] = jnp.exp(-jnp.sum(p * jnp.log(p + 1e-10)))


def kernel(inputs, W):
    flat = inputs.reshape(-1, _D)
    n = flat.shape[0]
    nblocks = n // _TB
    loss, q, perp, enc = pl.pallas_call(
        functools.partial(_vq_block, nblocks=nblocks, n_tokens=n),
        grid=(nblocks,),
        in_specs=[
            pl.BlockSpec((_TB, _D), lambda i: (i, 0)),
            pl.BlockSpec((_K, _D), lambda i: (0, 0)),
        ],
        out_specs=[
            pl.BlockSpec((1, 1), lambda i: (0, 0)),
            pl.BlockSpec((_TB, _D), lambda i: (i, 0)),
            pl.BlockSpec((1, 1), lambda i: (0, 0)),
            pl.BlockSpec((_TB, _K), lambda i: (i, 0)),
        ],
        out_shape=[
            jax.ShapeDtypeStruct((1, 1), jnp.float32),
            jax.ShapeDtypeStruct((n, _D), jnp.float32),
            jax.ShapeDtypeStruct((1, 1), jnp.float32),
            jax.ShapeDtypeStruct((n, _K), jnp.float32),
        ],
        scratch_shapes=[
            pltpu.SMEM((1, 1), jnp.float32),
            pltpu.VMEM((1, _K), jnp.float32),
        ],
    )(flat, W)
    return (loss[0, 0], q.reshape(inputs.shape), perp[0, 0], enc)


# speedup vs baseline: 2.9388x; 2.9388x over previous
import functools

import jax
import jax.numpy as jnp
from jax.experimental import pallas as pl
from jax.experimental.pallas import tpu as pltpu

_K = 1024
_D = 64
_TB = 512


def _vq_block(x_ref, w_ref, loss_ref, q_ref, perp_ref, enc_ref,
              loss_acc, hist_acc, *, nblocks, n_tokens):
    i = pl.program_id(0)

    @pl.when(i == 0)
    def _init():
        loss_acc[0, 0] = 0.0
        hist_acc[...] = jnp.zeros_like(hist_acc)

    x = x_ref[...]
    w = w_ref[...]
    x_sq = jnp.sum(x * x, axis=1, keepdims=True)
    w_sq = jnp.sum(w * w, axis=1)
    xw = jax.lax.dot_general(x, w, (((1,), (1,)), ((), ())))
    dist = x_sq + w_sq[None, :] - 2.0 * xw
    idx = jnp.argmin(dist, axis=1)
    iota = jax.lax.broadcasted_iota(jnp.int32, (x.shape[0], _K), 1)
    one_hot = (iota == idx[:, None]).astype(jnp.float32)
    enc_ref[...] = one_hot
    q = jax.lax.dot_general(one_hot, w, (((1,), (0,)), ((), ())))
    q_ref[...] = q
    d = q - x
    loss_acc[0, 0] += jnp.sum(d * d)
    hist_acc[...] += jnp.sum(one_hot, axis=0, keepdims=True)

    @pl.when(i == nblocks - 1)
    def _fin():
        m = loss_acc[0, 0] / (n_tokens * _D)
        loss_ref[...] = jnp.full((1, 1), m + 0.25 * m, jnp.float32)
        p = hist_acc[...] / n_tokens
        ent = jnp.sum(p * jnp.log(p + 1e-10), keepdims=True)
        perp_ref[...] = jnp.exp(-ent).reshape(1, 1)


def kernel(inputs, W):
    flat = inputs.reshape(-1, _D)
    n = flat.shape[0]
    nblocks = n // _TB
    loss, q, perp, enc = pl.pallas_call(
        functools.partial(_vq_block, nblocks=nblocks, n_tokens=n),
        grid=(nblocks,),
        in_specs=[
            pl.BlockSpec((_TB, _D), lambda i: (i, 0)),
            pl.BlockSpec((_K, _D), lambda i: (0, 0)),
        ],
        out_specs=[
            pl.BlockSpec((1, 1), lambda i: (0, 0)),
            pl.BlockSpec((_TB, _D), lambda i: (i, 0)),
            pl.BlockSpec((1, 1), lambda i: (0, 0)),
            pl.BlockSpec((_TB, _K), lambda i: (i, 0)),
        ],
        out_shape=[
            jax.ShapeDtypeStruct((1, 1), jnp.float32),
            jax.ShapeDtypeStruct((n, _D), jnp.float32),
            jax.ShapeDtypeStruct((1, 1), jnp.float32),
            jax.ShapeDtypeStruct((n, _K), jnp.float32),
        ],
        scratch_shapes=[
            pltpu.SMEM((1, 1), jnp.float32),
            pltpu.VMEM((1, _K), jnp.float32),
        ],
    )(flat, W)
    return (loss[0, 0], q.reshape(inputs.shape), perp[0, 0], enc)


# speedup vs baseline: 3.4721x; 1.1815x over previous
import functools

import jax
import jax.numpy as jnp
from jax.experimental import pallas as pl
from jax.experimental.pallas import tpu as pltpu

_K = 1024
_D = 64
_TB = 1024


def _vq_block(x_ref, w_ref, loss_ref, q_ref, perp_ref, enc_ref,
              loss_acc, hist_acc, wsq_acc, *, nblocks, n_tokens):
    i = pl.program_id(0)

    @pl.when(i == 0)
    def _init():
        loss_acc[0, 0] = 0.0
        hist_acc[...] = jnp.zeros_like(hist_acc)
        wv = w_ref[...]
        wsq_acc[...] = jnp.sum(wv * wv, axis=1)[None, :]

    x = x_ref[...]
    w = w_ref[...]
    x_sq = jnp.sum(x * x, axis=1, keepdims=True)
    w_sq = wsq_acc[...]
    xw = jax.lax.dot_general(x, w, (((1,), (1,)), ((), ())))
    dist = x_sq + w_sq - 2.0 * xw
    idx = jnp.argmin(dist, axis=1)
    iota = jax.lax.broadcasted_iota(jnp.int32, (x.shape[0], _K), 1)
    one_hot = (iota == idx[:, None]).astype(jnp.float32)
    enc_ref[...] = one_hot
    q = jax.lax.dot_general(one_hot, w, (((1,), (0,)), ((), ())))
    q_ref[...] = q
    d = q - x
    loss_acc[0, 0] += jnp.sum(d * d)
    hist_acc[...] += jnp.sum(one_hot, axis=0, keepdims=True)

    @pl.when(i == nblocks - 1)
    def _fin():
        m = loss_acc[0, 0] / (n_tokens * _D)
        loss_ref[...] = jnp.full((1, 1), m + 0.25 * m, jnp.float32)
        p = hist_acc[...] / n_tokens
        ent = jnp.sum(p * jnp.log(p + 1e-10), keepdims=True)
        perp_ref[...] = jnp.exp(-ent).reshape(1, 1)


def kernel(inputs, W):
    flat = inputs.reshape(-1, _D)
    n = flat.shape[0]
    nblocks = n // _TB
    loss, q, perp, enc = pl.pallas_call(
        functools.partial(_vq_block, nblocks=nblocks, n_tokens=n),
        grid=(nblocks,),
        in_specs=[
            pl.BlockSpec((_TB, _D), lambda i: (i, 0)),
            pl.BlockSpec((_K, _D), lambda i: (0, 0)),
        ],
        out_specs=[
            pl.BlockSpec((1, 1), lambda i: (0, 0)),
            pl.BlockSpec((_TB, _D), lambda i: (i, 0)),
            pl.BlockSpec((1, 1), lambda i: (0, 0)),
            pl.BlockSpec((_TB, _K), lambda i: (i, 0)),
        ],
        out_shape=[
            jax.ShapeDtypeStruct((1, 1), jnp.float32),
            jax.ShapeDtypeStruct((n, _D), jnp.float32),
            jax.ShapeDtypeStruct((1, 1), jnp.float32),
            jax.ShapeDtypeStruct((n, _K), jnp.float32),
        ],
        scratch_shapes=[
            pltpu.SMEM((1, 1), jnp.float32),
            pltpu.VMEM((1, _K), jnp.float32),
            pltpu.VMEM((1, _K), jnp.float32),
        ],
    )(flat, W)
    return (loss[0, 0], q.reshape(inputs.shape), perp[0, 0], enc)


# speedup vs baseline: 3.8172x; 1.0994x over previous
import functools

import jax
import jax.numpy as jnp
from jax.experimental import pallas as pl
from jax.experimental.pallas import tpu as pltpu

_K = 1024
_D = 64
_TB = 2048


def _vq_block(x_ref, w_ref, loss_ref, q_ref, perp_ref, enc_ref,
              loss_acc, hist_acc, wsq_acc, *, nblocks, n_tokens):
    i = pl.program_id(0)

    @pl.when(i == 0)
    def _init():
        loss_acc[0, 0] = 0.0
        hist_acc[...] = jnp.zeros_like(hist_acc)
        wv = w_ref[...]
        wsq_acc[...] = jnp.sum(wv * wv, axis=1)[None, :]

    x = x_ref[...]
    w = w_ref[...]
    x_sq = jnp.sum(x * x, axis=1, keepdims=True)
    w_sq = wsq_acc[...]
    xw = jax.lax.dot_general(x, w, (((1,), (1,)), ((), ())))
    dist = x_sq + w_sq - 2.0 * xw
    idx = jnp.argmin(dist, axis=1)
    iota = jax.lax.broadcasted_iota(jnp.int32, (x.shape[0], _K), 1)
    one_hot = (iota == idx[:, None]).astype(jnp.float32)
    enc_ref[...] = one_hot
    q = jax.lax.dot_general(one_hot, w, (((1,), (0,)), ((), ())))
    q_ref[...] = q
    d = q - x
    loss_acc[0, 0] += jnp.sum(d * d)
    hist_acc[...] += jnp.sum(one_hot, axis=0, keepdims=True)

    @pl.when(i == nblocks - 1)
    def _fin():
        m = loss_acc[0, 0] / (n_tokens * _D)
        loss_ref[...] = jnp.full((1, 1), m + 0.25 * m, jnp.float32)
        p = hist_acc[...] / n_tokens
        ent = jnp.sum(p * jnp.log(p + 1e-10), keepdims=True)
        perp_ref[...] = jnp.exp(-ent).reshape(1, 1)


def kernel(inputs, W):
    flat = inputs.reshape(-1, _D)
    n = flat.shape[0]
    nblocks = n // _TB
    loss, q, perp, enc = pl.pallas_call(
        functools.partial(_vq_block, nblocks=nblocks, n_tokens=n),
        grid=(nblocks,),
        in_specs=[
            pl.BlockSpec((_TB, _D), lambda i: (i, 0)),
            pl.BlockSpec((_K, _D), lambda i: (0, 0)),
        ],
        out_specs=[
            pl.BlockSpec((1, 1), lambda i: (0, 0)),
            pl.BlockSpec((_TB, _D), lambda i: (i, 0)),
            pl.BlockSpec((1, 1), lambda i: (0, 0)),
            pl.BlockSpec((_TB, _K), lambda i: (i, 0)),
        ],
        out_shape=[
            jax.ShapeDtypeStruct((1, 1), jnp.float32),
            jax.ShapeDtypeStruct((n, _D), jnp.float32),
            jax.ShapeDtypeStruct((1, 1), jnp.float32),
            jax.ShapeDtypeStruct((n, _K), jnp.float32),
        ],
        scratch_shapes=[
            pltpu.SMEM((1, 1), jnp.float32),
            pltpu.VMEM((1, _K), jnp.float32),
            pltpu.VMEM((1, _K), jnp.float32),
        ],
    )(flat, W)
    return (loss[0, 0], q.reshape(inputs.shape), perp[0, 0], enc)


# speedup vs baseline: 3.8714x; 1.0142x over previous
import functools

import jax
import jax.numpy as jnp
from jax.experimental import pallas as pl
from jax.experimental.pallas import tpu as pltpu

_K = 1024
_D = 64
_TB = 4096


def _vq_block(x_ref, w_ref, loss_ref, q_ref, perp_ref, enc_ref,
              loss_acc, hist_acc, wsq_acc, *, nblocks, n_tokens):
    i = pl.program_id(0)

    @pl.when(i == 0)
    def _init():
        loss_acc[0, 0] = 0.0
        hist_acc[...] = jnp.zeros_like(hist_acc)
        wv = w_ref[...]
        wsq_acc[...] = jnp.sum(wv * wv, axis=1)[None, :]

    x = x_ref[...]
    w = w_ref[...]
    x_sq = jnp.sum(x * x, axis=1, keepdims=True)
    w_sq = wsq_acc[...]
    xw = jax.lax.dot_general(x, w, (((1,), (1,)), ((), ())))
    dist = x_sq + w_sq - 2.0 * xw
    idx = jnp.argmin(dist, axis=1)
    iota = jax.lax.broadcasted_iota(jnp.int32, (x.shape[0], _K), 1)
    one_hot = (iota == idx[:, None]).astype(jnp.float32)
    enc_ref[...] = one_hot
    q = jax.lax.dot_general(one_hot, w, (((1,), (0,)), ((), ())))
    q_ref[...] = q
    d = q - x
    loss_acc[0, 0] += jnp.sum(d * d)
    hist_acc[...] += jnp.sum(one_hot, axis=0, keepdims=True)

    @pl.when(i == nblocks - 1)
    def _fin():
        m = loss_acc[0, 0] / (n_tokens * _D)
        loss_ref[...] = jnp.full((1, 1), m + 0.25 * m, jnp.float32)
        p = hist_acc[...] / n_tokens
        ent = jnp.sum(p * jnp.log(p + 1e-10), keepdims=True)
        perp_ref[...] = jnp.exp(-ent).reshape(1, 1)


def kernel(inputs, W):
    flat = inputs.reshape(-1, _D)
    n = flat.shape[0]
    nblocks = n // _TB
    loss, q, perp, enc = pl.pallas_call(
        functools.partial(_vq_block, nblocks=nblocks, n_tokens=n),
        grid=(nblocks,),
        in_specs=[
            pl.BlockSpec((_TB, _D), lambda i: (i, 0)),
            pl.BlockSpec((_K, _D), lambda i: (0, 0)),
        ],
        out_specs=[
            pl.BlockSpec((1, 1), lambda i: (0, 0)),
            pl.BlockSpec((_TB, _D), lambda i: (i, 0)),
            pl.BlockSpec((1, 1), lambda i: (0, 0)),
            pl.BlockSpec((_TB, _K), lambda i: (i, 0)),
        ],
        out_shape=[
            jax.ShapeDtypeStruct((1, 1), jnp.float32),
            jax.ShapeDtypeStruct((n, _D), jnp.float32),
            jax.ShapeDtypeStruct((1, 1), jnp.float32),
            jax.ShapeDtypeStruct((n, _K), jnp.float32),
        ],
        scratch_shapes=[
            pltpu.SMEM((1, 1), jnp.float32),
            pltpu.VMEM((1, _K), jnp.float32),
            pltpu.VMEM((1, _K), jnp.float32),
        ],
    )(flat, W)
    return (loss[0, 0], q.reshape(inputs.shape), perp[0, 0], enc)


# speedup vs baseline: 3.9935x; 1.0315x over previous
import functools

import jax
import jax.numpy as jnp
from jax.experimental import pallas as pl
from jax.experimental.pallas import tpu as pltpu

_K = 1024
_D = 64
_TB = 4096


def _vq_block(x_ref, w_ref, loss_ref, q_ref, perp_ref, enc_ref,
              loss_acc, hist_acc, wsq_acc, *, nblocks, n_tokens):
    i = pl.program_id(0)

    @pl.when(i == 0)
    def _init():
        loss_acc[0, 0] = 0.0
        hist_acc[...] = jnp.zeros_like(hist_acc)
        wv = w_ref[...]
        wsq_acc[...] = jnp.sum(wv * wv, axis=1)[None, :]

    x = x_ref[...]
    w = w_ref[...]
    x_sq = jnp.sum(x * x, axis=1, keepdims=True)
    w_sq = wsq_acc[...]
    xw = jax.lax.dot_general(x, w, (((1,), (1,)), ((), ())))
    dist = x_sq + w_sq - 2.0 * xw
    idx = jnp.argmin(dist, axis=1)
    iota = jax.lax.broadcasted_iota(jnp.int32, (x.shape[0], _K), 1)
    one_hot = (iota == idx[:, None]).astype(jnp.float32)
    enc_ref[...] = one_hot
    q = jax.lax.dot_general(one_hot, w, (((1,), (0,)), ((), ())))
    q_ref[...] = q
    d = q - x
    loss_acc[0, 0] += jnp.sum(d * d)
    ones_row = jnp.ones((1, x.shape[0]), jnp.float32)
    hist_acc[...] += jax.lax.dot_general(
        ones_row, one_hot, (((1,), (0,)), ((), ())))

    @pl.when(i == nblocks - 1)
    def _fin():
        m = loss_acc[0, 0] / (n_tokens * _D)
        loss_ref[...] = jnp.full((1, 1), m + 0.25 * m, jnp.float32)
        p = hist_acc[...] / n_tokens
        ent = jnp.sum(p * jnp.log(p + 1e-10), keepdims=True)
        perp_ref[...] = jnp.exp(-ent).reshape(1, 1)


def kernel(inputs, W):
    flat = inputs.reshape(-1, _D)
    n = flat.shape[0]
    nblocks = n // _TB
    loss, q, perp, enc = pl.pallas_call(
        functools.partial(_vq_block, nblocks=nblocks, n_tokens=n),
        grid=(nblocks,),
        in_specs=[
            pl.BlockSpec((_TB, _D), lambda i: (i, 0)),
            pl.BlockSpec((_K, _D), lambda i: (0, 0)),
        ],
        out_specs=[
            pl.BlockSpec((1, 1), lambda i: (0, 0)),
            pl.BlockSpec((_TB, _D), lambda i: (i, 0)),
            pl.BlockSpec((1, 1), lambda i: (0, 0)),
            pl.BlockSpec((_TB, _K), lambda i: (i, 0)),
        ],
        out_shape=[
            jax.ShapeDtypeStruct((1, 1), jnp.float32),
            jax.ShapeDtypeStruct((n, _D), jnp.float32),
            jax.ShapeDtypeStruct((1, 1), jnp.float32),
            jax.ShapeDtypeStruct((n, _K), jnp.float32),
        ],
        scratch_shapes=[
            pltpu.SMEM((1, 1), jnp.float32),
            pltpu.VMEM((1, _K), jnp.float32),
            pltpu.VMEM((1, _K), jnp.float32),
        ],
    )(flat, W)
    return (loss[0, 0], q.reshape(inputs.shape), perp[0, 0], enc)


# speedup vs baseline: 4.2066x; 1.0534x over previous
import functools

import jax
import jax.numpy as jnp
from jax.experimental import pallas as pl
from jax.experimental.pallas import tpu as pltpu

_K = 1024
_D = 64
_TB = 4096


def _vq_block(x_ref, w_ref, loss_ref, q_ref, perp_ref, enc_ref,
              loss_acc, hist_acc, wsq_acc, *, nblocks, n_tokens):
    i = pl.program_id(0)

    @pl.when(i == 0)
    def _init():
        loss_acc[0, 0] = 0.0
        hist_acc[...] = jnp.zeros_like(hist_acc)
        wv = w_ref[...]
        wsq_acc[...] = jnp.sum(wv * wv, axis=1)[None, :]

    x = x_ref[...].reshape(-1, _D)
    w = w_ref[...]
    x_sq = jnp.sum(x * x, axis=1, keepdims=True)
    w_sq = wsq_acc[...]
    xw = jax.lax.dot_general(x, w, (((1,), (1,)), ((), ())))
    dist = x_sq + w_sq - 2.0 * xw
    idx = jnp.argmin(dist, axis=1)
    iota = jax.lax.broadcasted_iota(jnp.int32, (x.shape[0], _K), 1)
    one_hot = (iota == idx[:, None]).astype(jnp.float32)
    enc_ref[...] = one_hot
    q = jax.lax.dot_general(one_hot, w, (((1,), (0,)), ((), ())))
    q_ref[...] = q.reshape(q_ref.shape)
    d = q - x
    loss_acc[0, 0] += jnp.sum(d * d)
    ones_row = jnp.ones((1, x.shape[0]), jnp.float32)
    hist_acc[...] += jax.lax.dot_general(
        ones_row, one_hot, (((1,), (0,)), ((), ())))

    @pl.when(i == nblocks - 1)
    def _fin():
        m = loss_acc[0, 0] / (n_tokens * _D)
        loss_ref[...] = jnp.full((1, 1), m + 0.25 * m, jnp.float32)
        p = hist_acc[...] / n_tokens
        ent = jnp.sum(p * jnp.log(p + 1e-10), keepdims=True)
        perp_ref[...] = jnp.exp(-ent).reshape(1, 1)


def kernel(inputs, W):
    b, s, d = inputs.shape
    n = b * s
    rows_per_block = _TB // s
    nblocks = n // _TB
    loss, q, perp, enc = pl.pallas_call(
        functools.partial(_vq_block, nblocks=nblocks, n_tokens=n),
        grid=(nblocks,),
        in_specs=[
            pl.BlockSpec((rows_per_block, s, d), lambda i: (i, 0, 0)),
            pl.BlockSpec((_K, _D), lambda i: (0, 0)),
        ],
        out_specs=[
            pl.BlockSpec((1, 1), lambda i: (0, 0)),
            pl.BlockSpec((rows_per_block, s, d), lambda i: (i, 0, 0)),
            pl.BlockSpec((1, 1), lambda i: (0, 0)),
            pl.BlockSpec((_TB, _K), lambda i: (i, 0)),
        ],
        out_shape=[
            jax.ShapeDtypeStruct((1, 1), jnp.float32),
            jax.ShapeDtypeStruct((b, s, d), jnp.float32),
            jax.ShapeDtypeStruct((1, 1), jnp.float32),
            jax.ShapeDtypeStruct((n, _K), jnp.float32),
        ],
        scratch_shapes=[
            pltpu.SMEM((1, 1), jnp.float32),
            pltpu.VMEM((1, _K), jnp.float32),
            pltpu.VMEM((1, _K), jnp.float32),
        ],
    )(inputs, W)
    return (loss[0, 0], q, perp[0, 0], enc)


# speedup vs baseline: 6.2840x; 1.4938x over previous
import functools

import jax
import jax.numpy as jnp
from jax.experimental import pallas as pl
from jax.experimental.pallas import tpu as pltpu

_K = 1024
_D = 64
_TB = 4096


def _vq_block(xt_ref, wt_ref, loss_ref, qt_ref, perp_ref, enc_ref,
              loss_acc, hist_acc, wsq_acc, *, nblocks, n_tokens):
    i = pl.program_id(0)
    wt = wt_ref[...]

    @pl.when(i == 0)
    def _init():
        loss_acc[0, 0] = 0.0
        hist_acc[...] = jnp.zeros_like(hist_acc)
        w = jnp.transpose(wt, (1, 0))
        wsq_acc[...] = jnp.sum(w * w, axis=1)[None, :]

    xc = xt_ref[...]
    nchunks, _, s = xc.shape
    x = jnp.transpose(xc, (0, 2, 1)).reshape(-1, _D)
    x_sq = jnp.sum(x * x, axis=1, keepdims=True)
    w_sq = wsq_acc[...]
    xw = jax.lax.dot_general(x, wt, (((1,), (0,)), ((), ())))
    dist = x_sq + w_sq - 2.0 * xw
    idx = jnp.argmin(dist, axis=1)
    iota = jax.lax.broadcasted_iota(jnp.int32, (x.shape[0], _K), 1)
    one_hot = (iota == idx[:, None]).astype(jnp.float32)
    enc_ref[...] = one_hot
    loss_new = loss_acc[0, 0]
    for c in range(nchunks):
        oh_c = one_hot[c * s:(c + 1) * s, :]
        q_tc = jax.lax.dot_general(wt, oh_c, (((1,), (1,)), ((), ())))
        qt_ref[c] = q_tc
        d = q_tc - xc[c]
        loss_new = loss_new + jnp.sum(d * d)
    loss_acc[0, 0] = loss_new
    ones_row = jnp.ones((1, x.shape[0]), jnp.float32)
    hist_acc[...] += jax.lax.dot_general(
        ones_row, one_hot, (((1,), (0,)), ((), ())))

    @pl.when(i == nblocks - 1)
    def _fin():
        m = loss_acc[0, 0] / (n_tokens * _D)
        loss_ref[...] = jnp.full((1, 1), m + 0.25 * m, jnp.float32)
        p = hist_acc[...] / n_tokens
        ent = jnp.sum(p * jnp.log(p + 1e-10), keepdims=True)
        perp_ref[...] = jnp.exp(-ent).reshape(1, 1)


def kernel(inputs, W):
    b, s, d = inputs.shape
    n = b * s
    xt = jnp.transpose(inputs, (0, 2, 1))
    wt = jnp.transpose(W, (1, 0))
    rows_per_block = _TB // s
    nblocks = n // _TB
    loss, qt, perp, enc = pl.pallas_call(
        functools.partial(_vq_block, nblocks=nblocks, n_tokens=n),
        grid=(nblocks,),
        in_specs=[
            pl.BlockSpec((rows_per_block, d, s), lambda i: (i, 0, 0)),
            pl.BlockSpec((_D, _K), lambda i: (0, 0)),
        ],
        out_specs=[
            pl.BlockSpec((1, 1), lambda i: (0, 0)),
            pl.BlockSpec((rows_per_block, d, s), lambda i: (i, 0, 0)),
            pl.BlockSpec((1, 1), lambda i: (0, 0)),
            pl.BlockSpec((_TB, _K), lambda i: (i, 0)),
        ],
        out_shape=[
            jax.ShapeDtypeStruct((1, 1), jnp.float32),
            jax.ShapeDtypeStruct((b, d, s), jnp.float32),
            jax.ShapeDtypeStruct((1, 1), jnp.float32),
            jax.ShapeDtypeStruct((n, _K), jnp.float32),
        ],
        scratch_shapes=[
            pltpu.SMEM((1, 1), jnp.float32),
            pltpu.VMEM((1, _K), jnp.float32),
            pltpu.VMEM((1, _K), jnp.float32),
        ],
    )(xt, wt)
    q = jnp.transpose(qt, (0, 2, 1))
    return (loss[0, 0], q, perp[0, 0], enc)
